# baseline (speedup 1.0000x reference)
import jax
import jax.numpy as jnp
from jax import lax
from jax.experimental import pallas as pl
from jax.experimental.pallas import tpu as pltpu
from jax.experimental.pallas import tpu_sc as plsc

_B = 4
_S = 2048
_D = 768
_NC = 2
_NS = 16
_NW = _NC * _NS
_SW = _S // _NW
_C = 8
_R = _C * _B
_NG = _SW // _C
_NBUF = 4
_J = _D // 16


def _body(idx_hbm, table_hbm, pos_hbm, out_hbm, idx_v, in_v, pos_v,
          sem_g, sem_o):
    wid = lax.axis_index("s") * _NC + lax.axis_index("c")
    s_base = wid * _SW

    pre = [
        pltpu.async_copy(idx_hbm.at[b, pl.ds(s_base, _SW)], idx_v.at[b],
                         sem_g)
        for b in range(_B)
    ]
    for cp in pre:
        cp.wait()

    gathers = {}
    stores = {}

    def start(i):
        slot = i % _NBUF
        gathers[i] = [
            pltpu.async_copy(
                table_hbm.at[idx_v.at[b, pl.ds(i * _C, _C)]],
                in_v.at[slot, b], sem_g)
            for b in range(_B)
        ] + [
            pltpu.async_copy(
                pos_hbm.at[pl.ds(s_base + i * _C, _C)], pos_v.at[slot],
                sem_g),
        ]

    for i in range(_NBUF - 1):
        start(i)
    for i in range(_NG):
        slot = i % _NBUF
        for cp in gathers.pop(i):
            cp.wait()

        @plsc.parallel_loop(0, _J, 1, unroll=2)
        def add_j(j, slot=slot):
            sl = pl.ds(j * 16, 16)
            for s in range(_C):
                p = pos_v[slot, s, sl]
                for b in range(_B):
                    in_v[slot, b, s, sl] = in_v[slot, b, s, sl] + p

        stores[i] = [
            pltpu.async_copy(
                in_v.at[slot],
                out_hbm.at[:, pl.ds(s_base + i * _C, _C)], sem_o)
        ]
        if i + _NBUF - 1 < _NG:
            if i >= 1:
                for cp in stores.pop(i - 1):
                    cp.wait()
            start(i + _NBUF - 1)
    for i in sorted(stores):
        for cp in stores.pop(i):
            cp.wait()


@jax.jit
def kernel(inputs, embed_table, pos_table):
    idx = inputs.astype(jnp.int32)
    mesh = plsc.VectorSubcoreMesh(core_axis_name="c", subcore_axis_name="s")
    out = pl.kernel(
        _body,
        out_type=jax.ShapeDtypeStruct((_B, _S, _D), jnp.float32),
        mesh=mesh,
        scratch_types=[
            pltpu.VMEM((_B, _SW), jnp.int32),
            pltpu.VMEM((_NBUF, _B, _C, _D), jnp.float32),
            pltpu.VMEM((_NBUF, _C, _D), jnp.float32),
            pltpu.SemaphoreType.DMA,
            pltpu.SemaphoreType.DMA,
        ],
    )(idx, embed_table, pos_table)
    return out


# speedup vs baseline: 1.1700x; 1.1700x over previous
import jax
import jax.numpy as jnp
from jax import lax
from jax.experimental import pallas as pl
from jax.experimental.pallas import tpu as pltpu
from jax.experimental.pallas import tpu_sc as plsc

_B = 4
_S = 2048
_D = 768
_NC = 2
_NS = 16
_NW = _NC * _NS
_SW = _S // _NW
_C = 8
_R = _C * _B
_NG = _SW // _C
_NBUF = 4
_J = _D // 16


def _body(idx_hbm, table_hbm, pos_hbm, out_hbm, idx_v, in_v, pos_v,
          sem_g, sem_o):
    wid = lax.axis_index("s") * _NC + lax.axis_index("c")
    s_base = wid * _SW

    pre = [
        pltpu.async_copy(idx_hbm.at[b, pl.ds(s_base, _SW)], idx_v.at[b],
                         sem_g)
        for b in range(_B)
    ]
    for cp in pre:
        cp.wait()

    gathers = {}
    stores = {}

    def start(i):
        slot = i % _NBUF
        gathers[i] = [
            pltpu.async_copy(
                table_hbm.at[idx_v.at[b, pl.ds(i * _C, _C)]],
                in_v.at[slot, b], sem_g)
            for b in range(_B)
        ] + [
            pltpu.async_copy(
                pos_hbm.at[pl.ds(s_base + i * _C, _C)], pos_v.at[slot],
                sem_g),
        ]

    for i in range(_NBUF - 1):
        start(i)
    for i in range(_NG):
        slot = i % _NBUF
        for cp in gathers.pop(i):
            cp.wait()

        def add_s(s, c, slot=slot):
            @plsc.parallel_loop(0, _J, 1, unroll=8)
            def add_j(j):
                sl = pl.ds(j * 16, 16)
                p = pos_v[slot, s, sl]
                for b in range(_B):
                    in_v[slot, b, s, sl] = in_v[slot, b, s, sl] + p
            return c

        lax.fori_loop(0, _C, add_s, 0)

        stores[i] = [
            pltpu.async_copy(
                in_v.at[slot],
                out_hbm.at[:, pl.ds(s_base + i * _C, _C)], sem_o)
        ]
        if i + _NBUF - 1 < _NG:
            if i >= 1:
                for cp in stores.pop(i - 1):
                    cp.wait()
            start(i + _NBUF - 1)
    for i in sorted(stores):
        for cp in stores.pop(i):
            cp.wait()


@jax.jit
def kernel(inputs, embed_table, pos_table):
    idx = inputs.astype(jnp.int32)
    mesh = plsc.VectorSubcoreMesh(core_axis_name="c", subcore_axis_name="s")
    out = pl.kernel(
        _body,
        out_type=jax.ShapeDtypeStruct((_B, _S, _D), jnp.float32),
        mesh=mesh,
        scratch_types=[
            pltpu.VMEM((_B, _SW), jnp.int32),
            pltpu.VMEM((_NBUF, _B, _C, _D), jnp.float32),
            pltpu.VMEM((_NBUF, _C, _D), jnp.float32),
            pltpu.SemaphoreType.DMA,
            pltpu.SemaphoreType.DMA,
        ],
    )(idx, embed_table, pos_table)
    return out
